# Initial kernel scaffold; baseline (speedup 1.0000x reference)
import jax
import jax.numpy as jnp
from jax.experimental import pallas as pl


def kernel(input_ids, position_ids, word_table, pos_table):
    raise NotImplementedError("write your pallas kernel here")


# speedup vs baseline: 1.3128x; 1.3128x over previous
import functools

import jax
import jax.numpy as jnp
from jax import lax
from jax.experimental import pallas as pl
from jax.experimental.pallas import tpu as pltpu
from jax.experimental.pallas import tpu_sc as plsc

_HIDDEN = 1024
_BATCH = 4
_SEQ = 8192
_TOK = _BATCH * _SEQ
_NC = 2
_NS = 16
_NW = _NC * _NS
_TPW = _TOK // _NW
_C = 16
_NCHUNK = _TPW // _C
_L = 16


def _emb_body(widx_hbm, pidx_hbm, wtab_hbm, ptab_hbm, out_hbm,
              idxw_v, idxp_v, wbuf, pbuf,
              wsem0, wsem1, psem0, psem1):
    wid = lax.axis_index("s") * _NC + lax.axis_index("c")
    base = wid * _TPW
    pltpu.sync_copy(widx_hbm.at[pl.ds(base, _TPW)], idxw_v)
    pltpu.sync_copy(pidx_hbm.at[pl.ds(base, _TPW)], idxp_v)

    wsems = (wsem0, wsem1)
    psems = (psem0, psem1)

    def _start(g, slot):
        pltpu.async_copy(wtab_hbm.at[idxw_v.at[pl.ds(g * _C, _C)]],
                         wbuf.at[slot], wsems[slot])
        pltpu.async_copy(ptab_hbm.at[idxp_v.at[pl.ds(g * _C, _C)]],
                         pbuf.at[slot], psems[slot])

    def _wait(g, slot):
        pltpu.make_async_copy(wtab_hbm.at[idxw_v.at[pl.ds(g * _C, _C)]],
                              wbuf.at[slot], wsems[slot]).wait()
        pltpu.make_async_copy(ptab_hbm.at[idxp_v.at[pl.ds(g * _C, _C)]],
                              pbuf.at[slot], psems[slot]).wait()

    def _chunk(g, slot):
        @pl.when(g + 1 < _NCHUNK)
        def _():
            _start(g + 1, (slot + 1) % 2)
        _wait(g, slot)
        wb = wbuf.at[slot]
        pb = pbuf.at[slot]

        def _row(r, carry):
            for j in range(_HIDDEN // _L):
                sl = (r, pl.ds(j * _L, _L))
                plsc.addupdate(wb.at[sl], pb[sl])
            return carry

        lax.fori_loop(0, _C, _row, 0)
        pltpu.sync_copy(wb, out_hbm.at[pl.ds(base + g * _C, _C)])

    _start(0, 0)

    def _pair(t, carry):
        _chunk(2 * t, 0)
        _chunk(2 * t + 1, 1)
        return carry

    lax.fori_loop(0, _NCHUNK // 2, _pair, 0)


@functools.partial(
    pl.kernel,
    out_type=jax.ShapeDtypeStruct((_TOK, _HIDDEN), jnp.float32),
    mesh=plsc.VectorSubcoreMesh(core_axis_name="c", subcore_axis_name="s",
                                num_cores=_NC, num_subcores=_NS),
    scratch_types=[
        pltpu.VMEM((_TPW,), jnp.int32),
        pltpu.VMEM((_TPW,), jnp.int32),
        pltpu.VMEM((2, _C, _HIDDEN), jnp.float32),
        pltpu.VMEM((2, _C, _HIDDEN), jnp.float32),
        pltpu.SemaphoreType.DMA,
        pltpu.SemaphoreType.DMA,
        pltpu.SemaphoreType.DMA,
        pltpu.SemaphoreType.DMA,
    ],
)
def _emb_kernel(widx_hbm, pidx_hbm, wtab_hbm, ptab_hbm, out_hbm,
                idxw_v, idxp_v, wbuf, pbuf, wsem0, wsem1, psem0, psem1):
    _emb_body(widx_hbm, pidx_hbm, wtab_hbm, ptab_hbm, out_hbm,
              idxw_v, idxp_v, wbuf, pbuf, wsem0, wsem1, psem0, psem1)


def kernel(input_ids, position_ids, word_table, pos_table):
    idw = jnp.transpose(input_ids).reshape(_TOK)
    idp = jnp.transpose(position_ids).reshape(_TOK)
    out = _emb_kernel(idw, idp, word_table, pos_table)
    return out.reshape(_SEQ, _BATCH, _HIDDEN)


# speedup vs baseline: 3.4094x; 2.5970x over previous
import functools

import jax
import jax.numpy as jnp
from jax import lax
from jax.experimental import pallas as pl
from jax.experimental.pallas import tpu as pltpu
from jax.experimental.pallas import tpu_sc as plsc

_HIDDEN = 1024
_BATCH = 4
_SEQ = 8192
_NC = 2
_NS = 16
_NW = _NC * _NS
_TPW = (_BATCH * _SEQ) // _NW
_C = 16
_NCHUNK = _TPW // _C
_L = 16


def _emb_body(wids_hbm, pids_hbm, wtab_hbm, ptab_hbm, out_hbm,
              idxw_v, idxp_v, wbuf, pbuf,
              wsem0, wsem1, psem0, psem1, ssem0, ssem1):
    wid = lax.axis_index("s") * _NC + lax.axis_index("c")
    b = wid & 3
    s0 = (wid >> 2) * _TPW
    pltpu.sync_copy(wids_hbm.at[b, pl.ds(s0, _TPW)], idxw_v)
    pltpu.sync_copy(pids_hbm.at[b, pl.ds(s0, _TPW)], idxp_v)

    wsems = (wsem0, wsem1)
    psems = (psem0, psem1)
    ssems = (ssem0, ssem1)

    def _start(g, slot):
        pltpu.async_copy(wtab_hbm.at[idxw_v.at[pl.ds(g * _C, _C)]],
                         wbuf.at[slot], wsems[slot])
        pltpu.async_copy(ptab_hbm.at[idxp_v.at[pl.ds(g * _C, _C)]],
                         pbuf.at[slot], psems[slot])

    def _wait(g, slot):
        pltpu.make_async_copy(wtab_hbm.at[idxw_v.at[pl.ds(g * _C, _C)]],
                              wbuf.at[slot], wsems[slot]).wait()
        pltpu.make_async_copy(ptab_hbm.at[idxp_v.at[pl.ds(g * _C, _C)]],
                              pbuf.at[slot], psems[slot]).wait()

    def _store(g, slot):
        return pltpu.async_copy(wbuf.at[slot],
                                out_hbm.at[pl.ds(s0 + g * _C, _C), b],
                                ssems[slot])

    def _wait_store(g, slot):
        pltpu.make_async_copy(wbuf.at[slot],
                              out_hbm.at[pl.ds(s0 + g * _C, _C), b],
                              ssems[slot]).wait()

    def _chunk(g, slot):
        @pl.when(g + 1 < _NCHUNK)
        def _():
            @pl.when(g >= 1)
            def _():
                _wait_store(g - 1, (slot + 1) % 2)
            _start(g + 1, (slot + 1) % 2)

        _wait(g, slot)
        wb = wbuf.at[slot]
        pb = pbuf.at[slot]

        @plsc.parallel_loop(0, _C * (_HIDDEN // _L), unroll=8)
        def _(i):
            r = i >> 6
            off = (i & 63) * _L
            plsc.addupdate(wb.at[r, pl.ds(off, _L)], pb[r, pl.ds(off, _L)])

        _store(g, slot)

    _start(0, 0)

    def _pair(t, carry):
        _chunk(2 * t, 0)
        _chunk(2 * t + 1, 1)
        return carry

    lax.fori_loop(0, _NCHUNK // 2, _pair, 0)
    _wait_store(_NCHUNK - 2, 0)
    _wait_store(_NCHUNK - 1, 1)


@functools.partial(
    pl.kernel,
    out_type=jax.ShapeDtypeStruct((_SEQ, _BATCH, _HIDDEN), jnp.float32),
    mesh=plsc.VectorSubcoreMesh(core_axis_name="c", subcore_axis_name="s",
                                num_cores=_NC, num_subcores=_NS),
    scratch_types=[
        pltpu.VMEM((_TPW,), jnp.int32),
        pltpu.VMEM((_TPW,), jnp.int32),
        pltpu.VMEM((2, _C, _HIDDEN), jnp.float32),
        pltpu.VMEM((2, _C, _HIDDEN), jnp.float32),
        pltpu.SemaphoreType.DMA,
        pltpu.SemaphoreType.DMA,
        pltpu.SemaphoreType.DMA,
        pltpu.SemaphoreType.DMA,
        pltpu.SemaphoreType.DMA,
        pltpu.SemaphoreType.DMA,
    ],
)
def _emb_kernel(wids_hbm, pids_hbm, wtab_hbm, ptab_hbm, out_hbm,
                idxw_v, idxp_v, wbuf, pbuf,
                wsem0, wsem1, psem0, psem1, ssem0, ssem1):
    _emb_body(wids_hbm, pids_hbm, wtab_hbm, ptab_hbm, out_hbm,
              idxw_v, idxp_v, wbuf, pbuf,
              wsem0, wsem1, psem0, psem1, ssem0, ssem1)


def kernel(input_ids, position_ids, word_table, pos_table):
    return _emb_kernel(input_ids, position_ids, word_table, pos_table)


# speedup vs baseline: 3.4802x; 1.0208x over previous
import functools

import jax
import jax.numpy as jnp
from jax import lax
from jax.experimental import pallas as pl
from jax.experimental.pallas import tpu as pltpu
from jax.experimental.pallas import tpu_sc as plsc

_HIDDEN = 1024
_BATCH = 4
_SEQ = 8192
_NC = 2
_NS = 16
_NW = _NC * _NS
_TPW = (_BATCH * _SEQ) // _NW
_C = 16
_NCHUNK = _TPW // _C
_NB = 3
_L = 16


def _emb_body(wids_hbm, pids_hbm, wtab_hbm, ptab_hbm, out_hbm,
              idxw_v, idxp_v, wbuf, pbuf, *sems):
    wsems = sems[0:_NB]
    psems = sems[_NB:2 * _NB]
    ssems = sems[2 * _NB:3 * _NB]

    wid = lax.axis_index("s") * _NC + lax.axis_index("c")
    b = wid & 3
    s0 = (wid >> 2) * _TPW
    pltpu.sync_copy(wids_hbm.at[b, pl.ds(s0, _TPW)], idxw_v)
    pltpu.sync_copy(pids_hbm.at[b, pl.ds(s0, _TPW)], idxp_v)

    def _start(g, slot):
        pltpu.async_copy(wtab_hbm.at[idxw_v.at[pl.ds(g * _C, _C)]],
                         wbuf.at[slot], wsems[slot])
        pltpu.async_copy(ptab_hbm.at[idxp_v.at[pl.ds(g * _C, _C)]],
                         pbuf.at[slot], psems[slot])

    def _wait(g, slot):
        pltpu.make_async_copy(wtab_hbm.at[idxw_v.at[pl.ds(g * _C, _C)]],
                              wbuf.at[slot], wsems[slot]).wait()
        pltpu.make_async_copy(ptab_hbm.at[idxp_v.at[pl.ds(g * _C, _C)]],
                              pbuf.at[slot], psems[slot]).wait()

    def _store(g, slot):
        return pltpu.async_copy(wbuf.at[slot],
                                out_hbm.at[pl.ds(s0 + g * _C, _C), b],
                                ssems[slot])

    def _wait_store(g, slot):
        pltpu.make_async_copy(wbuf.at[slot],
                              out_hbm.at[pl.ds(s0 + g * _C, _C), b],
                              ssems[slot]).wait()

    def _chunk(g, slot):
        @pl.when(g + _NB - 1 < _NCHUNK)
        def _():
            @pl.when(g >= 1)
            def _():
                _wait_store(g - 1, (slot + _NB - 1) % _NB)
            _start(g + _NB - 1, (slot + _NB - 1) % _NB)

        _wait(g, slot)
        wb = wbuf.at[slot]
        pb = pbuf.at[slot]

        @plsc.parallel_loop(0, _C * (_HIDDEN // _L), unroll=8)
        def _(i):
            r = i >> 6
            off = (i & 63) * _L
            plsc.addupdate(wb.at[r, pl.ds(off, _L)], pb[r, pl.ds(off, _L)])

        _store(g, slot)

    for g in range(_NB - 1):
        _start(g, g)

    def _triple(t, carry):
        for k in range(_NB):
            _chunk(_NB * t + k, k)
        return carry

    _NTR = (_NCHUNK - 1) // _NB
    lax.fori_loop(0, _NTR, _triple, 0)
    for g in range(_NB * _NTR, _NCHUNK):
        _chunk(g, g % _NB)
    for g in range(_NCHUNK - _NB, _NCHUNK):
        _wait_store(g, g % _NB)


@functools.partial(
    pl.kernel,
    out_type=jax.ShapeDtypeStruct((_SEQ, _BATCH, _HIDDEN), jnp.float32),
    mesh=plsc.VectorSubcoreMesh(core_axis_name="c", subcore_axis_name="s",
                                num_cores=_NC, num_subcores=_NS),
    scratch_types=[
        pltpu.VMEM((_TPW,), jnp.int32),
        pltpu.VMEM((_TPW,), jnp.int32),
        pltpu.VMEM((_NB, _C, _HIDDEN), jnp.float32),
        pltpu.VMEM((_NB, _C, _HIDDEN), jnp.float32),
    ] + [pltpu.SemaphoreType.DMA] * (3 * _NB),
)
def _emb_kernel(wids_hbm, pids_hbm, wtab_hbm, ptab_hbm, out_hbm,
                idxw_v, idxp_v, wbuf, pbuf, *sems):
    _emb_body(wids_hbm, pids_hbm, wtab_hbm, ptab_hbm, out_hbm,
              idxw_v, idxp_v, wbuf, pbuf, *sems)


def kernel(input_ids, position_ids, word_table, pos_table):
    return _emb_kernel(input_ids, position_ids, word_table, pos_table)
